# baseline (speedup 1.0000x reference)
import functools

import jax
import jax.numpy as jnp
from jax import lax
from jax.experimental import pallas as pl
from jax.experimental.pallas import tpu as pltpu
from jax.experimental.pallas import tpu_sc as plsc

NUM_WORKERS = 32
BATCH = 16384
SEQ = 200
D = 64
VOCAB = 1000000
ROWS_PER_W = BATCH // NUM_WORKERS
CHUNK = 4
IDX_PER_CHUNK = CHUNK * SEQ
N_CHUNKS = ROWS_PER_W // CHUNK
GSLICE = 80
N_GATHERS = IDX_PER_CHUNK // GSLICE

TB = 128
NBLK = -(-VOCAB // TB)
NFULL = VOCAB // TB
DT_STEPS = -(-NBLK // NUM_WORKERS)
BLK_ELEMS = TB * D
TAIL_ROW0 = VOCAB - TB


def _detile_body(table_hbm, tail_hbm, out_hbm, bin0, bin1, bout0, bout1,
                 semr0, semr1, semw0, semw1):
    wid = lax.axis_index("s") * 2 + lax.axis_index("c")
    bins = (bin0, bin1)
    bouts = (bout0, bout1)
    semr = (semr0, semr1)
    semw = (semw0, semw1)

    def block_of(k):
        return wid + NUM_WORKERS * k

    def start_read(k, p):
        b = block_of(k)

        @pl.when(b < NFULL)
        def _():
            pltpu.async_copy(table_hbm.at[:, pl.ds(b * TB, TB)],
                             bins[p], semr[p])

        @pl.when(b == NFULL)
        def _():
            pltpu.async_copy(tail_hbm.at[:, :], bins[p], semr[p])

    iota64 = lax.iota(jnp.int32, 16) * D

    def finish(k, p, first):
        b = block_of(k)

        @pl.when(b < NFULL)
        def _():
            pltpu.make_async_copy(table_hbm.at[:, pl.ds(0, TB)],
                                  bins[p], semr[p]).wait()

        @pl.when(b == NFULL)
        def _():
            pltpu.make_async_copy(tail_hbm.at[:, :], bins[p], semr[p]).wait()

        @pl.when(b < NBLK)
        def _():
            if not first:
                pltpu.make_async_copy(
                    bouts[p], out_hbm.at[pl.ds(0, BLK_ELEMS)],
                    semw[p]).wait()

            def mv(rg, carry):
                idx = iota64 + (16 * rg * D)
                for c in range(D):
                    v = bins[p][c, pl.ds(16 * rg, 16)]
                    plsc.store_scatter(bouts[p], [idx + c], v)
                return carry

            lax.fori_loop(0, TB // 16, mv, 0)
            row0 = jnp.where(b == NFULL, TAIL_ROW0, b * TB)
            pltpu.async_copy(bouts[p],
                             out_hbm.at[pl.ds(row0 * D, BLK_ELEMS)],
                             semw[p])

    start_read(0, 0)
    start_read(1, 1)
    finish(0, 0, True)
    start_read(2, 0)
    finish(1, 1, True)
    start_read(3, 1)

    def body(k2, carry):
        k0 = 2 * k2
        finish(k0, 0, False)
        start_read(k0 + 2, 0)
        finish(k0 + 1, 1, False)
        start_read(k0 + 3, 1)
        return carry

    lax.fori_loop(1, (DT_STEPS + 1) // 2, body, 0)

    pltpu.make_async_copy(bouts[0], out_hbm.at[pl.ds(0, BLK_ELEMS)],
                          semw[0]).wait()
    pltpu.make_async_copy(bouts[1], out_hbm.at[pl.ds(0, BLK_ELEMS)],
                          semw[1]).wait()


def _sc_body(labels_hbm, table_hbm, out_hbm,
             idx_e, idx_o, rows_e, rows_o, out_stage, sem_e, sem_o):
    wid = lax.axis_index("s") * 2 + lax.axis_index("c")
    w_idx_base = wid * ROWS_PER_W * SEQ
    w_row_base = wid * ROWS_PER_W

    def load_idx(chunk, idx_v):
        pltpu.sync_copy(
            labels_hbm.at[pl.ds(w_idx_base + chunk * IDX_PER_CHUNK,
                                IDX_PER_CHUNK)],
            idx_v)

    def fire_gathers(idx_v, rows_v, sem):
        for j in range(N_GATHERS):
            pltpu.async_copy(
                table_hbm.at[idx_v.at[pl.ds(j * GSLICE, GSLICE)]],
                rows_v.at[pl.ds(j * GSLICE, GSLICE)],
                sem)

    def drain(idx_v, rows_v, sem):
        for j in range(N_GATHERS):
            pltpu.make_async_copy(
                table_hbm.at[idx_v.at[pl.ds(j * GSLICE, GSLICE)]],
                rows_v.at[pl.ds(j * GSLICE, GSLICE)],
                sem).wait()

    inv = jnp.float32(1.0 / SEQ)

    def accumulate(rows_v):
        for i in range(CHUNK):
            base = i * SEQ

            def body(j, acc):
                r = base + 4 * j
                a0, a1, a2, a3 = acc
                for k in range(4):
                    a0 = a0 + rows_v[r + k, pl.ds(0, 16)]
                    a1 = a1 + rows_v[r + k, pl.ds(16, 16)]
                    a2 = a2 + rows_v[r + k, pl.ds(32, 16)]
                    a3 = a3 + rows_v[r + k, pl.ds(48, 16)]
                return (a0, a1, a2, a3)

            z = jnp.zeros((16,), jnp.float32)
            a0, a1, a2, a3 = lax.fori_loop(0, SEQ // 4, body, (z, z, z, z))
            out_stage[i, pl.ds(0, 16)] = a0 * inv
            out_stage[i, pl.ds(16, 16)] = a1 * inv
            out_stage[i, pl.ds(32, 16)] = a2 * inv
            out_stage[i, pl.ds(48, 16)] = a3 * inv

    def store_out(chunk):
        pltpu.sync_copy(out_stage,
                        out_hbm.at[pl.ds(w_row_base + chunk * CHUNK, CHUNK)])

    load_idx(0, idx_e)
    fire_gathers(idx_e, rows_e, sem_e)

    def outer(g0, carry):
        c0 = 2 * g0
        c1 = 2 * g0 + 1

        load_idx(c1, idx_o)
        fire_gathers(idx_o, rows_o, sem_o)

        drain(idx_e, rows_e, sem_e)
        accumulate(rows_e)
        store_out(c0)

        @pl.when(g0 < N_CHUNKS // 2 - 1)
        def _():
            load_idx(c0 + 2, idx_e)
            fire_gathers(idx_e, rows_e, sem_e)

        drain(idx_o, rows_o, sem_o)
        accumulate(rows_o)
        store_out(c1)
        return carry

    lax.fori_loop(0, N_CHUNKS // 2, outer, 0)


def kernel(labels, table):
    labels_flat = labels.reshape(BATCH * SEQ).astype(jnp.int32)
    mesh = plsc.VectorSubcoreMesh(core_axis_name="c", subcore_axis_name="s")
    detile = pl.kernel(
        _detile_body,
        out_type=jax.ShapeDtypeStruct((VOCAB * D,), jnp.float32),
        mesh=mesh,
        scratch_types=[
            pltpu.VMEM((D, TB), jnp.float32),
            pltpu.VMEM((D, TB), jnp.float32),
            pltpu.VMEM((BLK_ELEMS,), jnp.float32),
            pltpu.VMEM((BLK_ELEMS,), jnp.float32),
            pltpu.SemaphoreType.DMA,
            pltpu.SemaphoreType.DMA,
            pltpu.SemaphoreType.DMA,
            pltpu.SemaphoreType.DMA,
        ],
        compiler_params=pltpu.CompilerParams(use_tc_tiling_on_sc=True,
                                             needs_layout_passes=False),
    )
    tail = table[TAIL_ROW0:, :].T
    table_lin = detile(table.T, tail).reshape(VOCAB, D)
    f = pl.kernel(
        _sc_body,
        out_type=jax.ShapeDtypeStruct((BATCH, D), jnp.float32),
        mesh=mesh,
        scratch_types=[
            pltpu.VMEM((IDX_PER_CHUNK,), jnp.int32),
            pltpu.VMEM((IDX_PER_CHUNK,), jnp.int32),
            pltpu.VMEM((IDX_PER_CHUNK, D), jnp.float32),
            pltpu.VMEM((IDX_PER_CHUNK, D), jnp.float32),
            pltpu.VMEM((CHUNK, D), jnp.float32),
            pltpu.SemaphoreType.DMA,
            pltpu.SemaphoreType.DMA,
        ],
        compiler_params=pltpu.CompilerParams(use_tc_tiling_on_sc=False),
    )
    return f(labels_flat, table_lin)


# speedup vs baseline: 1.4533x; 1.4533x over previous
import functools

import jax
import jax.numpy as jnp
from jax import lax
from jax.experimental import pallas as pl
from jax.experimental.pallas import tpu as pltpu
from jax.experimental.pallas import tpu_sc as plsc

NUM_WORKERS = 32
BATCH = 16384
SEQ = 200
D = 64
VOCAB = 1000000
ROWS_PER_W = BATCH // NUM_WORKERS
CHUNK = 4
IDX_PER_CHUNK = CHUNK * SEQ
N_CHUNKS = ROWS_PER_W // CHUNK
GSLICE = 80
N_GATHERS = IDX_PER_CHUNK // GSLICE

TB = 128
NBLK = -(-VOCAB // TB)
NFULL = VOCAB // TB
DT_STEPS = -(-NBLK // NUM_WORKERS)
BLK_ELEMS = TB * D
TAIL_ROW0 = VOCAB - TB


def _detile_body(table_hbm, tail_hbm, out_hbm, bin0, bin1, bout0, bout1,
                 semr0, semr1, semw0, semw1):
    wid = lax.axis_index("s") * 2 + lax.axis_index("c")
    bins = (bin0, bin1)
    bouts = (bout0, bout1)
    semr = (semr0, semr1)
    semw = (semw0, semw1)

    def block_of(k):
        return wid + NUM_WORKERS * k

    def start_read(k, p):
        b = block_of(k)

        @pl.when(b < NFULL)
        def _():
            pltpu.async_copy(table_hbm.at[:, pl.ds(b * TB, TB)],
                             bins[p], semr[p])

        @pl.when(b == NFULL)
        def _():
            pltpu.async_copy(tail_hbm.at[:, :], bins[p], semr[p])

    iota64 = lax.iota(jnp.int32, 16) * D

    def finish(k, p, first):
        b = block_of(k)

        @pl.when(b < NFULL)
        def _():
            pltpu.make_async_copy(table_hbm.at[:, pl.ds(0, TB)],
                                  bins[p], semr[p]).wait()

        @pl.when(b == NFULL)
        def _():
            pltpu.make_async_copy(tail_hbm.at[:, :], bins[p], semr[p]).wait()

        @pl.when(b < NBLK)
        def _():
            if not first:
                pltpu.make_async_copy(
                    bouts[p], out_hbm.at[pl.ds(0, BLK_ELEMS)],
                    semw[p]).wait()

            def mv(rg, carry):
                idx = iota64 + (16 * rg * D)
                for c in range(D):
                    v = bins[p][c, pl.ds(16 * rg, 16)]
                    plsc.store_scatter(bouts[p], [idx + c], v)
                return carry

            lax.fori_loop(0, TB // 16, mv, 0)
            row0 = jnp.where(b == NFULL, TAIL_ROW0, b * TB)
            pltpu.async_copy(bouts[p],
                             out_hbm.at[pl.ds(row0 * D, BLK_ELEMS)],
                             semw[p])

    start_read(0, 0)
    start_read(1, 1)
    finish(0, 0, True)
    start_read(2, 0)
    finish(1, 1, True)
    start_read(3, 1)

    def body(k2, carry):
        k0 = 2 * k2
        finish(k0, 0, False)
        start_read(k0 + 2, 0)
        finish(k0 + 1, 1, False)
        start_read(k0 + 3, 1)
        return carry

    lax.fori_loop(1, (DT_STEPS + 1) // 2, body, 0)

    pltpu.make_async_copy(bouts[0], out_hbm.at[pl.ds(0, BLK_ELEMS)],
                          semw[0]).wait()
    pltpu.make_async_copy(bouts[1], out_hbm.at[pl.ds(0, BLK_ELEMS)],
                          semw[1]).wait()


def _sc_body(labels_hbm, table_hbm, out_hbm,
             idx_e, idx_o, rows_e, rows_o, out_stage, sem_e, sem_o):
    wid = lax.axis_index("s") * 2 + lax.axis_index("c")
    w_idx_base = wid * ROWS_PER_W * SEQ
    w_row_base = wid * ROWS_PER_W

    def load_idx(chunk, idx_v):
        pltpu.sync_copy(
            labels_hbm.at[pl.ds(w_idx_base + chunk * IDX_PER_CHUNK,
                                IDX_PER_CHUNK)],
            idx_v)

    def fire_gathers(idx_v, rows_v, sem):
        for j in range(N_GATHERS):
            pltpu.async_copy(
                table_hbm.at[idx_v.at[pl.ds(j * GSLICE, GSLICE)]],
                rows_v.at[pl.ds(j * GSLICE, GSLICE)],
                sem)

    def drain(idx_v, rows_v, sem):
        for j in range(N_GATHERS):
            pltpu.make_async_copy(
                table_hbm.at[idx_v.at[pl.ds(j * GSLICE, GSLICE)]],
                rows_v.at[pl.ds(j * GSLICE, GSLICE)],
                sem).wait()

    inv = jnp.float32(1.0 / SEQ)
    iota2 = lax.iota(jnp.int32, 16) * 2

    def accumulate(rows_v):
        for i in range(CHUNK):
            base = i * SEQ

            def body(j, acc):
                r = base + 4 * j
                e0, o0, e1, o1 = acc
                for k in range(4):
                    a, b = plsc.unpack(rows_v[r + k, pl.ds(0, 32)],
                                       format=plsc.PackFormat.INTERLEAVED)
                    e0 = e0 + a
                    o0 = o0 + b
                    a, b = plsc.unpack(rows_v[r + k, pl.ds(32, 32)],
                                       format=plsc.PackFormat.INTERLEAVED)
                    e1 = e1 + a
                    o1 = o1 + b
                return (e0, o0, e1, o1)

            z = jnp.zeros((16,), jnp.float32)
            e0, o0, e1, o1 = lax.fori_loop(0, SEQ // 4, body, (z, z, z, z))
            si = jnp.full((16,), i, jnp.int32)
            plsc.store_scatter(out_stage, [si, iota2], e0 * inv)
            plsc.store_scatter(out_stage, [si, iota2 + 1], o0 * inv)
            plsc.store_scatter(out_stage, [si, iota2 + 32], e1 * inv)
            plsc.store_scatter(out_stage, [si, iota2 + 33], o1 * inv)

    def store_out(chunk):
        pltpu.sync_copy(out_stage,
                        out_hbm.at[pl.ds(w_row_base + chunk * CHUNK, CHUNK)])

    load_idx(0, idx_e)
    fire_gathers(idx_e, rows_e, sem_e)

    def outer(g0, carry):
        c0 = 2 * g0
        c1 = 2 * g0 + 1

        load_idx(c1, idx_o)
        fire_gathers(idx_o, rows_o, sem_o)

        drain(idx_e, rows_e, sem_e)
        accumulate(rows_e)
        store_out(c0)

        @pl.when(g0 < N_CHUNKS // 2 - 1)
        def _():
            load_idx(c0 + 2, idx_e)
            fire_gathers(idx_e, rows_e, sem_e)

        drain(idx_o, rows_o, sem_o)
        accumulate(rows_o)
        store_out(c1)
        return carry

    lax.fori_loop(0, N_CHUNKS // 2, outer, 0)


def kernel(labels, table):
    labels_flat = labels.reshape(BATCH * SEQ).astype(jnp.int32)
    mesh = plsc.VectorSubcoreMesh(core_axis_name="c", subcore_axis_name="s")
    detile = pl.kernel(
        _detile_body,
        out_type=jax.ShapeDtypeStruct((VOCAB * D,), jnp.float32),
        mesh=mesh,
        scratch_types=[
            pltpu.VMEM((D, TB), jnp.float32),
            pltpu.VMEM((D, TB), jnp.float32),
            pltpu.VMEM((BLK_ELEMS,), jnp.float32),
            pltpu.VMEM((BLK_ELEMS,), jnp.float32),
            pltpu.SemaphoreType.DMA,
            pltpu.SemaphoreType.DMA,
            pltpu.SemaphoreType.DMA,
            pltpu.SemaphoreType.DMA,
        ],
        compiler_params=pltpu.CompilerParams(use_tc_tiling_on_sc=True,
                                             needs_layout_passes=False),
    )
    table_bf = table.astype(jnp.bfloat16)
    f = pl.kernel(
        _sc_body,
        out_type=jax.ShapeDtypeStruct((BATCH, D), jnp.float32),
        mesh=mesh,
        scratch_types=[
            pltpu.VMEM((IDX_PER_CHUNK,), jnp.int32),
            pltpu.VMEM((IDX_PER_CHUNK,), jnp.int32),
            pltpu.VMEM((IDX_PER_CHUNK, D), jnp.bfloat16),
            pltpu.VMEM((IDX_PER_CHUNK, D), jnp.bfloat16),
            pltpu.VMEM((CHUNK, D), jnp.float32),
            pltpu.SemaphoreType.DMA,
            pltpu.SemaphoreType.DMA,
        ],
        compiler_params=pltpu.CompilerParams(use_tc_tiling_on_sc=False,
                                             needs_layout_passes=False),
    )
    return f(labels_flat, table_bf)
